# Initial kernel scaffold; baseline (speedup 1.0000x reference)
import functools

import jax
import jax.numpy as jnp
from jax import lax
from jax.experimental import pallas as pl
from jax.experimental.pallas import tpu as pltpu
from jax.experimental.pallas import tpu_sc as plsc

EPS = 1e-7
K = 80
WUI = 80
D = 128


def _pack_edges(col, val, row, col_offset=0):
    c = col.shape[0] // K
    cb = (col.astype(jnp.int32) + col_offset).reshape(c, K)
    vb = lax.bitcast_convert_type(val.astype(jnp.float32), jnp.int32).reshape(c, K)
    rb = row.astype(jnp.int32).reshape(c, K)
    return jnp.stack([cb, vb, rb], axis=1)


def _scale_rows(gbuf, ebuf, width):
    for e in range(K):
        vbits = plsc.load_gather(
            ebuf, [jnp.full((16,), 1, jnp.int32), jnp.full((16,), e, jnp.int32)])
        vb = plsc.bitcast(vbits, jnp.float32)
        for j in range(width // 16):
            sl = pl.ds(j * 16, 16)
            gbuf[e, sl] = gbuf[e, sl] * vb


def _sc_spmm_ui(tbl_a, tbl_b, packed, zeros, n):
    nchunks_total = packed.shape[0]
    nchunks = nchunks_total // 16
    rows_sub = n // 16
    mesh = plsc.VectorSubcoreMesh(core_axis_name="c", subcore_axis_name="s")

    @functools.partial(
        pl.kernel,
        out_type=(jax.ShapeDtypeStruct((n, WUI), jnp.float32),
                  jax.ShapeDtypeStruct((n, WUI), jnp.float32)),
        mesh=mesh,
        scratch_types=[
            pltpu.VMEM((3, K), jnp.int32),
            pltpu.VMEM((K, WUI), jnp.float32),
            pltpu.VMEM_SHARED((n, WUI), jnp.float32),
            pltpu.SemaphoreType.DMA,
        ],
    )
    def k(tbl_a_h, tbl_b_h, packed_h, zeros_h, out_a_h, out_b_h,
          ebuf, gbuf, acc, sem):
        cid = lax.axis_index("c")
        sid = lax.axis_index("s")
        my_rows = pl.ds(sid * rows_sub, rows_sub)
        pltpu.sync_copy(zeros_h, acc.at[my_rows])
        plsc.subcore_barrier()

        def run(tbl_h, out_h):
            def chunk(t, _):
                chunk_id = sid * nchunks + t
                pltpu.sync_copy(packed_h.at[chunk_id], ebuf)
                pltpu.async_copy(tbl_h.at[ebuf.at[0]], gbuf, sem).wait()
                _scale_rows(gbuf, ebuf, WUI)
                pltpu.sync_copy(gbuf, acc.at[ebuf.at[2]], add=True)
                return 0

            lax.fori_loop(0, nchunks, chunk, 0)
            plsc.subcore_barrier()
            pltpu.sync_copy(acc.at[my_rows], out_h.at[my_rows])

        @pl.when(cid == 0)
        def _():
            run(tbl_a_h, out_a_h)

        @pl.when(cid == 1)
        def _():
            run(tbl_b_h, out_b_h)

    return k(tbl_a, tbl_b, packed, zeros)


def _sc_spmm_ii(tbl, packed, zeros, n):
    nchunks_total = packed.shape[0]
    nchunks = nchunks_total // 32
    rows_sub = n // 16
    mesh = plsc.VectorSubcoreMesh(core_axis_name="c", subcore_axis_name="s")

    @functools.partial(
        pl.kernel,
        out_type=(jax.ShapeDtypeStruct((n, D), jnp.float32),
                  jax.ShapeDtypeStruct((n, D), jnp.float32)),
        mesh=mesh,
        scratch_types=[
            pltpu.VMEM((3, K), jnp.int32),
            pltpu.VMEM((K, D), jnp.float32),
            pltpu.VMEM_SHARED((n, D), jnp.float32),
            pltpu.SemaphoreType.DMA,
        ],
    )
    def k(tbl_h, packed_h, zeros_h, out0_h, out1_h, ebuf, gbuf, acc, sem):
        cid = lax.axis_index("c")
        sid = lax.axis_index("s")
        my_rows = pl.ds(sid * rows_sub, rows_sub)
        pltpu.sync_copy(zeros_h, acc.at[my_rows])
        plsc.subcore_barrier()

        def chunk(t, _):
            chunk_id = (cid * 16 + sid) * nchunks + t
            pltpu.sync_copy(packed_h.at[chunk_id], ebuf)
            pltpu.async_copy(tbl_h.at[ebuf.at[0]], gbuf, sem).wait()
            _scale_rows(gbuf, ebuf, D)
            pltpu.sync_copy(gbuf, acc.at[ebuf.at[2]], add=True)
            return 0

        lax.fori_loop(0, nchunks, chunk, 0)
        plsc.subcore_barrier()

        @pl.when(cid == 0)
        def _():
            pltpu.sync_copy(acc.at[my_rows], out0_h.at[my_rows])

        @pl.when(cid == 1)
        def _():
            pltpu.sync_copy(acc.at[my_rows], out1_h.at[my_rows])

    return k(tbl, packed, zeros)


def _pre_tables(x):
    n2 = jnp.sum(x * x, axis=-1, keepdims=True)
    nrm = jnp.maximum(jnp.sqrt(n2), EPS)
    xh = (jnp.tanh(nrm) / nrm) * x
    nh2 = jnp.sum(xh * xh, axis=-1, keepdims=True)
    lam = 2.0 / jnp.maximum(1.0 - nh2, EPS)
    y = lam * xh
    pad = jnp.zeros((x.shape[0], WUI - 65), x.dtype)
    pad1 = jnp.zeros((x.shape[0], WUI - 64), x.dtype)
    ta = jnp.concatenate([y[:, :64], lam - 1.0, pad], axis=-1)
    tb = jnp.concatenate([y[:, 64:], pad1], axis=-1)
    return ta, tb


def _agg_from_halves(a, b):
    num = jnp.concatenate([a[:, :64], b[:, :64]], axis=-1)
    den = a[:, 64:65]
    den = jnp.where(jnp.abs(den) < EPS, EPS, den)
    t = num / den
    nt = jnp.maximum(jnp.sqrt(jnp.sum(t * t, axis=-1, keepdims=True)), EPS)
    u = jnp.minimum(nt, 1.0 - 1e-5)
    att = 0.5 * jnp.log((1.0 + u) / (1.0 - u))
    return (0.5 * att / nt) * t


def _tc_pre1(xu, xi, blk=1000):
    n = xu.shape[0] + xi.shape[0]
    g_half = xu.shape[0] // blk
    grid = n // blk

    def body(xu_ref, xi_ref, ta_ref, tb_ref):
        g = pl.program_id(0)

        def emit(x):
            ta, tb = _pre_tables(x)
            ta_ref[...] = ta
            tb_ref[...] = tb

        @pl.when(g < g_half)
        def _():
            emit(xu_ref[...])

        @pl.when(g >= g_half)
        def _():
            emit(xi_ref[...])

    return pl.pallas_call(
        body,
        grid=(grid,),
        in_specs=[
            pl.BlockSpec((blk, D), lambda g: (jnp.minimum(g, g_half - 1), 0)),
            pl.BlockSpec((blk, D), lambda g: (jnp.maximum(g - g_half, 0), 0)),
        ],
        out_specs=[
            pl.BlockSpec((blk, WUI), lambda g: (g, 0)),
            pl.BlockSpec((blk, WUI), lambda g: (g, 0)),
        ],
        out_shape=(jax.ShapeDtypeStruct((n, WUI), jnp.float32),
                   jax.ShapeDtypeStruct((n, WUI), jnp.float32)),
    )(xu, xi)


def _tc_mid(w, out_a, out_b, ii0, ii1, blk=1000):
    n = out_a.shape[0]
    g_half = (n // 2) // blk
    grid = n // blk

    def body(w_ref, a_ref, b_ref, i0_ref, i1_ref, h_ref, ta_ref, tb_ref):
        g = pl.program_id(0)
        agg = _agg_from_halves(a_ref[...], b_ref[...])

        def emit(h):
            h_ref[...] = h
            ta, tb = _pre_tables(h)
            ta_ref[...] = ta
            tb_ref[...] = tb

        @pl.when(g < g_half)
        def _():
            emit(w_ref[0] * agg)

        @pl.when(g >= g_half)
        def _():
            emit(w_ref[1] * agg + w_ref[2] * (i0_ref[...] + i1_ref[...]))

    return pl.pallas_call(
        body,
        grid=(grid,),
        in_specs=[
            pl.BlockSpec(memory_space=pltpu.SMEM),
            pl.BlockSpec((blk, WUI), lambda g: (g, 0)),
            pl.BlockSpec((blk, WUI), lambda g: (g, 0)),
            pl.BlockSpec((blk, D), lambda g: (jnp.maximum(g - g_half, 0), 0)),
            pl.BlockSpec((blk, D), lambda g: (jnp.maximum(g - g_half, 0), 0)),
        ],
        out_specs=[
            pl.BlockSpec((blk, D), lambda g: (g, 0)),
            pl.BlockSpec((blk, WUI), lambda g: (g, 0)),
            pl.BlockSpec((blk, WUI), lambda g: (g, 0)),
        ],
        out_shape=(jax.ShapeDtypeStruct((n, D), jnp.float32),
                   jax.ShapeDtypeStruct((n, WUI), jnp.float32),
                   jax.ShapeDtypeStruct((n, WUI), jnp.float32)),
    )(w, out_a, out_b, ii0, ii1)


def _tc_final(w, out_a, out_b, ii0, ii1, h1, blk=1000):
    n = out_a.shape[0]
    nu = n // 2
    g_half = nu // blk
    grid = g_half

    def body(w_ref, au_ref, ai_ref, bu_ref, bi_ref, i0_ref, i1_ref,
             h1u_ref, h1i_ref, hu_ref, hi_ref):
        aggu = _agg_from_halves(au_ref[...], bu_ref[...])
        aggi = _agg_from_halves(ai_ref[...], bi_ref[...])
        hu_ref[...] = h1u_ref[...] + w_ref[0] * aggu
        hi_ref[...] = (h1i_ref[...] + w_ref[1] * aggi
                       + w_ref[2] * (i0_ref[...] + i1_ref[...]))

    return pl.pallas_call(
        body,
        grid=(grid,),
        in_specs=[
            pl.BlockSpec(memory_space=pltpu.SMEM),
            pl.BlockSpec((blk, WUI), lambda g: (g, 0)),
            pl.BlockSpec((blk, WUI), lambda g: (g + g_half, 0)),
            pl.BlockSpec((blk, WUI), lambda g: (g, 0)),
            pl.BlockSpec((blk, WUI), lambda g: (g + g_half, 0)),
            pl.BlockSpec((blk, D), lambda g: (g, 0)),
            pl.BlockSpec((blk, D), lambda g: (g, 0)),
            pl.BlockSpec((blk, D), lambda g: (g, 0)),
            pl.BlockSpec((blk, D), lambda g: (g + g_half, 0)),
        ],
        out_specs=[
            pl.BlockSpec((blk, D), lambda g: (g, 0)),
            pl.BlockSpec((blk, D), lambda g: (g, 0)),
        ],
        out_shape=(jax.ShapeDtypeStruct((nu, D), jnp.float32),
                   jax.ShapeDtypeStruct((nu, D), jnp.float32)),
    )(w, out_a, out_a, out_b, out_b, ii0, ii1, h1, h1)


def kernel(x_user, x_item, adj_ui_row, adj_ui_col, adj_ui_val,
           adj_ii_row, adj_ii_col, adj_ii_val,
           w_user_ui, w_item_ui, w_item_ii):
    nu, d = x_user.shape
    ni = x_item.shape[0]
    n = nu + ni
    assert d == D

    w = jnp.concatenate([w_user_ui.astype(jnp.float32),
                         w_item_ui.astype(jnp.float32),
                         w_item_ii.astype(jnp.float32)])
    packed_ui = _pack_edges(adj_ui_col, adj_ui_val, adj_ui_row)
    packed_ii = _pack_edges(adj_ii_col, adj_ii_val, adj_ii_row)
    packed_ii2 = _pack_edges(adj_ii_col, adj_ii_val, adj_ii_row, col_offset=nu)
    z_ui = jnp.zeros((n // 16, WUI), jnp.float32)
    z_ii = jnp.zeros((ni // 16, D), jnp.float32)

    ta1, tb1 = _tc_pre1(x_user, x_item)
    ua1, ub1 = _sc_spmm_ui(ta1, tb1, packed_ui, z_ui, n)
    ii1a, ii1b = _sc_spmm_ii(x_item, packed_ii, z_ii, ni)
    h1, ta2, tb2 = _tc_mid(w, ua1, ub1, ii1a, ii1b)

    ua2, ub2 = _sc_spmm_ui(ta2, tb2, packed_ui, z_ui, n)
    ii2a, ii2b = _sc_spmm_ii(h1, packed_ii2, z_ii, ni)
    hu, hi = _tc_final(w, ua2, ub2, ii2a, ii2b, h1)
    return (hu, hi)


# speedup vs baseline: 5.7634x; 5.7634x over previous
import functools

import jax
import jax.numpy as jnp
from jax import lax
from jax.experimental import pallas as pl
from jax.experimental.pallas import tpu as pltpu
from jax.experimental.pallas import tpu_sc as plsc

EPS = 1e-7
K = 80
WUI = 80
D = 128


def _pack_edges(col, val, row, col_offset=0):
    c = col.shape[0] // K
    cb = (col.astype(jnp.int32) + col_offset).reshape(c, K)
    vb = lax.bitcast_convert_type(val.astype(jnp.float32), jnp.int32).reshape(c, K)
    rb = row.astype(jnp.int32).reshape(c, K)
    return jnp.stack([cb, vb, rb], axis=1)


def _row_split(n):
    r0 = ((n // 16 + 7) // 8) * 8
    return r0, n - 15 * r0


def _split_copy(sid, r0, r_last, fn):
    @pl.when(sid < 15)
    def _():
        fn(sid * r0, r0)

    @pl.when(sid == 15)
    def _():
        fn(15 * r0, r_last)


_BCAST_DN = lax.GatherDimensionNumbers(
    offset_dims=(), collapsed_slice_dims=(0,), start_index_map=(0,))


def _bcast_lane(v16, i):
    return lax.gather(v16, jnp.full((16, 1), i, jnp.int32), _BCAST_DN, (1,),
                      mode=lax.GatherScatterMode.PROMISE_IN_BOUNDS)


def _scale_rows(gbuf, ebuf, width):
    for g in range(K // 16):
        vals16 = plsc.bitcast(ebuf[1, pl.ds(g * 16, 16)], jnp.float32)
        for i in range(16):
            vb = _bcast_lane(vals16, i)
            e = g * 16 + i
            for j in range(width // 16):
                sl = pl.ds(j * 16, 16)
                gbuf[e, sl] = gbuf[e, sl] * vb


def _sc_spmm_ui(tbl_a, tbl_b, packed, zeros, n):
    nchunks_total = packed.shape[0]
    nchunks = nchunks_total // 16
    r0, r_last = _row_split(n)
    mesh = plsc.VectorSubcoreMesh(core_axis_name="c", subcore_axis_name="s")

    @functools.partial(
        pl.kernel,
        out_type=(jax.ShapeDtypeStruct((n, WUI), jnp.float32),
                  jax.ShapeDtypeStruct((n, WUI), jnp.float32)),
        mesh=mesh,
        compiler_params=pltpu.CompilerParams(needs_layout_passes=False, use_tc_tiling_on_sc=False),
        scratch_types=[
            pltpu.VMEM((3, K), jnp.int32),
            pltpu.VMEM((K, WUI), jnp.float32),
            pltpu.VMEM_SHARED((n, WUI), jnp.float32),
            pltpu.SemaphoreType.DMA,
        ],
    )
    def k(tbl_a_h, tbl_b_h, packed_h, zeros_h, out_a_h, out_b_h,
          ebuf, gbuf, acc, sem):
        cid = lax.axis_index("c")
        sid = lax.axis_index("s")
        _split_copy(sid, r0, r_last, lambda off, sz: pltpu.sync_copy(
            zeros_h.at[pl.ds(0, sz)], acc.at[pl.ds(off, sz)]))
        plsc.subcore_barrier()

        def run(tbl_h, out_h):
            def chunk(t, _):
                chunk_id = sid * nchunks + t
                pltpu.sync_copy(packed_h.at[chunk_id], ebuf)
                pltpu.async_copy(tbl_h.at[ebuf.at[0]], gbuf, sem).wait()
                _scale_rows(gbuf, ebuf, WUI)
                pltpu.sync_copy(gbuf, acc.at[ebuf.at[2]], add=True)
                return 0

            lax.fori_loop(0, nchunks, chunk, 0)
            plsc.subcore_barrier()
            _split_copy(sid, r0, r_last, lambda off, sz: pltpu.sync_copy(
                acc.at[pl.ds(off, sz)], out_h.at[pl.ds(off, sz)]))

        @pl.when(cid == 0)
        def _():
            run(tbl_a_h, out_a_h)

        @pl.when(cid == 1)
        def _():
            run(tbl_b_h, out_b_h)

    return k(tbl_a, tbl_b, packed, zeros)


def _sc_spmm_ii(tbl, packed, zeros, n):
    nchunks_total = packed.shape[0]
    nchunks = nchunks_total // 32
    r0, r_last = _row_split(n)
    mesh = plsc.VectorSubcoreMesh(core_axis_name="c", subcore_axis_name="s")

    @functools.partial(
        pl.kernel,
        out_type=(jax.ShapeDtypeStruct((n, D), jnp.float32),
                  jax.ShapeDtypeStruct((n, D), jnp.float32)),
        mesh=mesh,
        compiler_params=pltpu.CompilerParams(needs_layout_passes=False, use_tc_tiling_on_sc=False),
        scratch_types=[
            pltpu.VMEM((3, K), jnp.int32),
            pltpu.VMEM((K, D), jnp.float32),
            pltpu.VMEM_SHARED((n, D), jnp.float32),
            pltpu.SemaphoreType.DMA,
        ],
    )
    def k(tbl_h, packed_h, zeros_h, out0_h, out1_h, ebuf, gbuf, acc, sem):
        cid = lax.axis_index("c")
        sid = lax.axis_index("s")
        _split_copy(sid, r0, r_last, lambda off, sz: pltpu.sync_copy(
            zeros_h.at[pl.ds(0, sz)], acc.at[pl.ds(off, sz)]))
        plsc.subcore_barrier()

        def chunk(t, _):
            chunk_id = (cid * 16 + sid) * nchunks + t
            pltpu.sync_copy(packed_h.at[chunk_id], ebuf)
            pltpu.async_copy(tbl_h.at[ebuf.at[0]], gbuf, sem).wait()
            _scale_rows(gbuf, ebuf, D)
            pltpu.sync_copy(gbuf, acc.at[ebuf.at[2]], add=True)
            return 0

        lax.fori_loop(0, nchunks, chunk, 0)
        plsc.subcore_barrier()

        @pl.when(cid == 0)
        def _():
            _split_copy(sid, r0, r_last, lambda off, sz: pltpu.sync_copy(
                acc.at[pl.ds(off, sz)], out0_h.at[pl.ds(off, sz)]))

        @pl.when(cid == 1)
        def _():
            _split_copy(sid, r0, r_last, lambda off, sz: pltpu.sync_copy(
                acc.at[pl.ds(off, sz)], out1_h.at[pl.ds(off, sz)]))

    return k(tbl, packed, zeros)


def _pre_tables(x):
    n2 = jnp.sum(x * x, axis=-1, keepdims=True)
    nrm = jnp.maximum(jnp.sqrt(n2), EPS)
    xh = (jnp.tanh(nrm) / nrm) * x
    nh2 = jnp.sum(xh * xh, axis=-1, keepdims=True)
    lam = 2.0 / jnp.maximum(1.0 - nh2, EPS)
    y = lam * xh
    pad = jnp.zeros((x.shape[0], WUI - 65), x.dtype)
    pad1 = jnp.zeros((x.shape[0], WUI - 64), x.dtype)
    ta = jnp.concatenate([y[:, :64], lam - 1.0, pad], axis=-1)
    tb = jnp.concatenate([y[:, 64:], pad1], axis=-1)
    return ta, tb


def _agg_from_halves(a, b):
    num = jnp.concatenate([a[:, :64], b[:, :64]], axis=-1)
    den = a[:, 64:65]
    den = jnp.where(jnp.abs(den) < EPS, EPS, den)
    t = num / den
    nt = jnp.maximum(jnp.sqrt(jnp.sum(t * t, axis=-1, keepdims=True)), EPS)
    u = jnp.minimum(nt, 1.0 - 1e-5)
    att = 0.5 * jnp.log((1.0 + u) / (1.0 - u))
    return (0.5 * att / nt) * t


def _tc_pre1(xu, xi, blk=1000):
    n = xu.shape[0] + xi.shape[0]
    g_half = xu.shape[0] // blk
    grid = n // blk

    def body(xu_ref, xi_ref, ta_ref, tb_ref):
        g = pl.program_id(0)

        def emit(x):
            ta, tb = _pre_tables(x)
            ta_ref[...] = ta
            tb_ref[...] = tb

        @pl.when(g < g_half)
        def _():
            emit(xu_ref[...])

        @pl.when(g >= g_half)
        def _():
            emit(xi_ref[...])

    return pl.pallas_call(
        body,
        grid=(grid,),
        in_specs=[
            pl.BlockSpec((blk, D), lambda g: (jnp.minimum(g, g_half - 1), 0)),
            pl.BlockSpec((blk, D), lambda g: (jnp.maximum(g - g_half, 0), 0)),
        ],
        out_specs=[
            pl.BlockSpec((blk, WUI), lambda g: (g, 0)),
            pl.BlockSpec((blk, WUI), lambda g: (g, 0)),
        ],
        out_shape=(jax.ShapeDtypeStruct((n, WUI), jnp.float32),
                   jax.ShapeDtypeStruct((n, WUI), jnp.float32)),
    )(xu, xi)


def _tc_mid(w, out_a, out_b, ii0, ii1, blk=1000):
    n = out_a.shape[0]
    g_half = (n // 2) // blk
    grid = n // blk

    def body(w_ref, a_ref, b_ref, i0_ref, i1_ref, h_ref, ta_ref, tb_ref):
        g = pl.program_id(0)
        agg = _agg_from_halves(a_ref[...], b_ref[...])

        def emit(h):
            h_ref[...] = h
            ta, tb = _pre_tables(h)
            ta_ref[...] = ta
            tb_ref[...] = tb

        @pl.when(g < g_half)
        def _():
            emit(w_ref[0] * agg)

        @pl.when(g >= g_half)
        def _():
            emit(w_ref[1] * agg + w_ref[2] * (i0_ref[...] + i1_ref[...]))

    return pl.pallas_call(
        body,
        grid=(grid,),
        in_specs=[
            pl.BlockSpec(memory_space=pltpu.SMEM),
            pl.BlockSpec((blk, WUI), lambda g: (g, 0)),
            pl.BlockSpec((blk, WUI), lambda g: (g, 0)),
            pl.BlockSpec((blk, D), lambda g: (jnp.maximum(g - g_half, 0), 0)),
            pl.BlockSpec((blk, D), lambda g: (jnp.maximum(g - g_half, 0), 0)),
        ],
        out_specs=[
            pl.BlockSpec((blk, D), lambda g: (g, 0)),
            pl.BlockSpec((blk, WUI), lambda g: (g, 0)),
            pl.BlockSpec((blk, WUI), lambda g: (g, 0)),
        ],
        out_shape=(jax.ShapeDtypeStruct((n, D), jnp.float32),
                   jax.ShapeDtypeStruct((n, WUI), jnp.float32),
                   jax.ShapeDtypeStruct((n, WUI), jnp.float32)),
    )(w, out_a, out_b, ii0, ii1)


def _tc_final(w, out_a, out_b, ii0, ii1, h1, blk=1000):
    n = out_a.shape[0]
    nu = n // 2
    g_half = nu // blk
    grid = g_half

    def body(w_ref, au_ref, ai_ref, bu_ref, bi_ref, i0_ref, i1_ref,
             h1u_ref, h1i_ref, hu_ref, hi_ref):
        aggu = _agg_from_halves(au_ref[...], bu_ref[...])
        aggi = _agg_from_halves(ai_ref[...], bi_ref[...])
        hu_ref[...] = h1u_ref[...] + w_ref[0] * aggu
        hi_ref[...] = (h1i_ref[...] + w_ref[1] * aggi
                       + w_ref[2] * (i0_ref[...] + i1_ref[...]))

    return pl.pallas_call(
        body,
        grid=(grid,),
        in_specs=[
            pl.BlockSpec(memory_space=pltpu.SMEM),
            pl.BlockSpec((blk, WUI), lambda g: (g, 0)),
            pl.BlockSpec((blk, WUI), lambda g: (g + g_half, 0)),
            pl.BlockSpec((blk, WUI), lambda g: (g, 0)),
            pl.BlockSpec((blk, WUI), lambda g: (g + g_half, 0)),
            pl.BlockSpec((blk, D), lambda g: (g, 0)),
            pl.BlockSpec((blk, D), lambda g: (g, 0)),
            pl.BlockSpec((blk, D), lambda g: (g, 0)),
            pl.BlockSpec((blk, D), lambda g: (g + g_half, 0)),
        ],
        out_specs=[
            pl.BlockSpec((blk, D), lambda g: (g, 0)),
            pl.BlockSpec((blk, D), lambda g: (g, 0)),
        ],
        out_shape=(jax.ShapeDtypeStruct((nu, D), jnp.float32),
                   jax.ShapeDtypeStruct((nu, D), jnp.float32)),
    )(w, out_a, out_a, out_b, out_b, ii0, ii1, h1, h1)


def kernel(x_user, x_item, adj_ui_row, adj_ui_col, adj_ui_val,
           adj_ii_row, adj_ii_col, adj_ii_val,
           w_user_ui, w_item_ui, w_item_ii):
    nu, d = x_user.shape
    ni = x_item.shape[0]
    n = nu + ni
    assert d == D

    w = jnp.concatenate([w_user_ui.astype(jnp.float32),
                         w_item_ui.astype(jnp.float32),
                         w_item_ii.astype(jnp.float32)])
    packed_ui = _pack_edges(adj_ui_col, adj_ui_val, adj_ui_row)
    packed_ii = _pack_edges(adj_ii_col, adj_ii_val, adj_ii_row)
    packed_ii2 = _pack_edges(adj_ii_col, adj_ii_val, adj_ii_row, col_offset=nu)
    z_ui = jnp.zeros((_row_split(n)[0], WUI), jnp.float32)
    z_ii = jnp.zeros((_row_split(ni)[0], D), jnp.float32)

    ta1, tb1 = _tc_pre1(x_user, x_item)
    ua1, ub1 = _sc_spmm_ui(ta1, tb1, packed_ui, z_ui, n)
    ii1a, ii1b = _sc_spmm_ii(x_item, packed_ii, z_ii, ni)
    h1, ta2, tb2 = _tc_mid(w, ua1, ub1, ii1a, ii1b)

    ua2, ub2 = _sc_spmm_ui(ta2, tb2, packed_ui, z_ui, n)
    ii2a, ii2b = _sc_spmm_ii(h1, packed_ii2, z_ii, ni)
    hu, hi = _tc_final(w, ua2, ub2, ii2a, ii2b, h1)
    return (hu, hi)
